# Initial kernel scaffold; baseline (speedup 1.0000x reference)
import jax
import jax.numpy as jnp
from jax.experimental import pallas as pl


def kernel(z, edge_label_index):
    raise NotImplementedError("write your pallas kernel here")


# speedup vs baseline: 1.7948x; 1.7948x over previous
import jax
import jax.numpy as jnp
from jax import lax
from jax.experimental import pallas as pl
from jax.experimental.pallas import tpu as pltpu
from jax.experimental.pallas import tpu_sc as plsc

D = 16
C = 128
NC, NS = 2, 16
NW = NC * NS


def _sc_body(z_hbm, src_hbm, dst_hbm, out_hbm,
             sidx, didx, srows, drows, obuf, sem_s, sem_d):
    wid = lax.axis_index("s") * NC + lax.axis_index("c")
    nchunks = out_hbm.shape[0] // C
    n_t = (nchunks - wid + NW - 1) // NW

    def chunk(t, carry):
        base = (wid + t * NW) * C
        pltpu.sync_copy(src_hbm.at[pl.ds(base, C)], sidx)
        pltpu.sync_copy(dst_hbm.at[pl.ds(base, C)], didx)
        cp_s = pltpu.async_copy(z_hbm.at[sidx], srows, sem_s)
        cp_d = pltpu.async_copy(z_hbm.at[didx], drows, sem_d)
        cp_s.wait()
        cp_d.wait()

        def edge(e, c2):
            dvec = drows[e, :]
            svec = srows[e, :]
            for i in range(D):
                obuf[e, pl.ds(i * D, D)] = dvec * svec[i]
            return c2

        lax.fori_loop(0, C, edge, 0, unroll=False)
        pltpu.sync_copy(obuf, out_hbm.at[pl.ds(base, C)])
        return carry

    lax.fori_loop(0, n_t, chunk, 0, unroll=False)


def kernel(z, edge_label_index):
    e = edge_label_index.shape[1]
    src = edge_label_index[0]
    dst = edge_label_index[1]
    mesh = plsc.VectorSubcoreMesh(core_axis_name="c", subcore_axis_name="s")
    out = pl.kernel(
        _sc_body,
        out_type=jax.ShapeDtypeStruct((e, D * D), jnp.float32),
        mesh=mesh,
        scratch_types=[
            pltpu.VMEM((C,), jnp.int32),
            pltpu.VMEM((C,), jnp.int32),
            pltpu.VMEM((C, D), jnp.float32),
            pltpu.VMEM((C, D), jnp.float32),
            pltpu.VMEM((C, D * D), jnp.float32),
            pltpu.SemaphoreType.DMA,
            pltpu.SemaphoreType.DMA,
        ],
        compiler_params=pltpu.CompilerParams(use_tc_tiling_on_sc=False),
    )(z, src, dst)
    return out.reshape(e, 1, D, D)


# speedup vs baseline: 2.6152x; 1.4571x over previous
import jax
import jax.numpy as jnp
from jax import lax
from jax.experimental import pallas as pl
from jax.experimental.pallas import tpu as pltpu
from jax.experimental.pallas import tpu_sc as plsc

D = 16
C = 64
NC, NS = 2, 16
NW = NC * NS
EPW = 800000 // NW
NFULL = EPW // C
TAIL = EPW - NFULL * C


def _sc_body(z_hbm, src_hbm, dst_hbm, out_hbm,
             sidx0, sidx1, didx0, didx1, sr0, sr1, dr0, dr1, ob0, ob1,
             tsidx, tdidx,
             sem_is0, sem_is1, sem_id0, sem_id1,
             sem_gs0, sem_gs1, sem_gd0, sem_gd1,
             sem_o0, sem_o1):
    sidx = (sidx0, sidx1)
    didx = (didx0, didx1)
    srows = (sr0, sr1)
    drows = (dr0, dr1)
    obuf = (ob0, ob1)
    sem_is = (sem_is0, sem_is1)
    sem_id = (sem_id0, sem_id1)
    sem_gs = (sem_gs0, sem_gs1)
    sem_gd = (sem_gd0, sem_gd1)
    sem_o = (sem_o0, sem_o1)

    wid = lax.axis_index("s") * NC + lax.axis_index("c")
    w0 = wid * EPW

    def idx_cp(t, b):
        base = w0 + t * C
        return (
            pltpu.make_async_copy(src_hbm.at[pl.ds(base, C)], sidx[b],
                                  sem_is[b]),
            pltpu.make_async_copy(dst_hbm.at[pl.ds(base, C)], didx[b],
                                  sem_id[b]),
        )

    def gather_s(b):
        return pltpu.make_async_copy(z_hbm.at[sidx[b]], srows[b], sem_gs[b])

    def gather_d(b):
        return pltpu.make_async_copy(z_hbm.at[didx[b]], drows[b], sem_gd[b])

    def out_cp(t, b):
        return pltpu.make_async_copy(
            obuf[b], out_hbm.at[pl.ds(w0 + t * C, C)], sem_o[b])

    def compute(b, n):
        def edge(e, c2):
            dvec = drows[b][e, pl.ds(0, D)]
            svec = srows[b][e, pl.ds(0, D)]
            for i in range(D):
                obuf[b][e, pl.ds(i * D, D)] = dvec * svec[i]
            return c2
        lax.fori_loop(0, n, edge, 0, unroll=2)

    def start2(cps):
        cps[0].start()
        cps[1].start()

    def wait2(cps):
        cps[0].wait()
        cps[1].wait()

    cp0 = idx_cp(0, 0)
    start2(cp0)
    wait2(cp0)
    gather_s(0).start()
    gather_d(0).start()
    start2(idx_cp(1, 1))

    def step(t, b):
        gather_s(b).wait()
        gather_d(b).wait()
        pl.when(t <= NFULL - 3)(lambda: start2(idx_cp(t + 2, b)))
        pl.when(t >= 2)(lambda: out_cp(t - 2, b).wait())
        compute(b, C)
        out_cp(t, b).start()
        wait2(idx_cp(t + 1, 1 - b))
        gather_s(1 - b).start()
        gather_d(1 - b).start()

    def pair(tt, c):
        step(2 * tt, 0)
        step(2 * tt + 1, 1)
        return c

    lax.fori_loop(0, (NFULL - 2) // 2, pair, 0, unroll=False)

    t_pen = NFULL - 2
    gather_s(0).wait()
    gather_d(0).wait()
    out_cp(t_pen - 2, 0).wait()
    compute(0, C)
    out_cp(t_pen, 0).start()
    wait2(idx_cp(NFULL - 1, 1))
    gather_s(1).start()
    gather_d(1).start()

    t_last = NFULL - 1
    gather_s(1).wait()
    gather_d(1).wait()
    out_cp(t_last - 2, 1).wait()
    compute(1, C)
    out_cp(t_last, 1).start()

    tail0 = w0 + NFULL * C
    pltpu.sync_copy(src_hbm.at[pl.ds(tail0, TAIL)], tsidx)
    pltpu.sync_copy(dst_hbm.at[pl.ds(tail0, TAIL)], tdidx)
    gts = pltpu.make_async_copy(z_hbm.at[tsidx],
                                srows[0].at[pl.ds(0, TAIL)], sem_gs[0])
    gtd = pltpu.make_async_copy(z_hbm.at[tdidx],
                                drows[0].at[pl.ds(0, TAIL)], sem_gd[0])
    gts.start()
    gtd.start()
    gts.wait()
    gtd.wait()
    compute(0, TAIL)
    out_cp(t_pen, 0).wait()
    tail_out = pltpu.make_async_copy(obuf[0].at[pl.ds(0, TAIL)],
                                     out_hbm.at[pl.ds(tail0, TAIL)], sem_o[0])
    tail_out.start()

    out_cp(t_last, 1).wait()
    tail_out.wait()


def kernel(z, edge_label_index):
    e = edge_label_index.shape[1]
    src = edge_label_index[0]
    dst = edge_label_index[1]
    zt = jnp.pad(z, ((0, 0), (0, 128 - D)))
    mesh = plsc.VectorSubcoreMesh(core_axis_name="c", subcore_axis_name="s")
    out = pl.kernel(
        _sc_body,
        out_type=jax.ShapeDtypeStruct((e, D * D), jnp.float32),
        mesh=mesh,
        scratch_types=[
            pltpu.VMEM((C,), jnp.int32),
            pltpu.VMEM((C,), jnp.int32),
            pltpu.VMEM((C,), jnp.int32),
            pltpu.VMEM((C,), jnp.int32),
            pltpu.VMEM((C, 128), jnp.float32),
            pltpu.VMEM((C, 128), jnp.float32),
            pltpu.VMEM((C, 128), jnp.float32),
            pltpu.VMEM((C, 128), jnp.float32),
            pltpu.VMEM((C, D * D), jnp.float32),
            pltpu.VMEM((C, D * D), jnp.float32),
            pltpu.VMEM((TAIL,), jnp.int32),
            pltpu.VMEM((TAIL,), jnp.int32),
            pltpu.SemaphoreType.DMA,
            pltpu.SemaphoreType.DMA,
            pltpu.SemaphoreType.DMA,
            pltpu.SemaphoreType.DMA,
            pltpu.SemaphoreType.DMA,
            pltpu.SemaphoreType.DMA,
            pltpu.SemaphoreType.DMA,
            pltpu.SemaphoreType.DMA,
            pltpu.SemaphoreType.DMA,
            pltpu.SemaphoreType.DMA,
        ],
    )(zt, src, dst)
    return out.reshape(e, 1, D, D)


# speedup vs baseline: 4.3844x; 1.6765x over previous
import jax
import jax.numpy as jnp
from jax import lax
from jax.experimental import pallas as pl
from jax.experimental.pallas import tpu as pltpu
from jax.experimental.pallas import tpu_sc as plsc

D = 16
B = 128
H = 64
NC, NS = 2, 16
NW = NC * NS
NBLK = 800000 // B
NP0 = NBLK // NW
NEXTRA = NBLK - NP0 * NW


def _sc_body(z_hbm, src_hbm, dst_hbm, out_hbm,
             sidx0, sidx1, didx0, didx1, sr0, sr1, dr0, dr1, ob0, ob1,
             sem_is0, sem_is1, sem_id0, sem_id1,
             sem_gs0, sem_gs1, sem_gd0, sem_gd1,
             sem_o0, sem_o1):
    sidx = (sidx0, sidx1)
    didx = (didx0, didx1)
    rows = (sr0, sr1)
    drows = (dr0, dr1)
    obuf = (ob0, ob1)
    sem_is = (sem_is0, sem_is1)
    sem_id = (sem_id0, sem_id1)
    sem_gs = (sem_gs0, sem_gs1)
    sem_gd = (sem_gd0, sem_gd1)
    sem_o = (sem_o0, sem_o1)

    wid = lax.axis_index("s") * NC + lax.axis_index("c")

    def base(p):
        return (wid + p * NW) * B

    def idx_cp(p, b):
        return (
            pltpu.make_async_copy(src_hbm.at[pl.ds(base(p), B)], sidx[b],
                                  sem_is[b]),
            pltpu.make_async_copy(dst_hbm.at[pl.ds(base(p), B)], didx[b],
                                  sem_id[b]),
        )

    def gather(b, h):
        isl = pl.ds(h * H, H)
        return (
            pltpu.make_async_copy(z_hbm.at[sidx[b].at[isl]], rows[h],
                                  sem_gs[h]),
            pltpu.make_async_copy(z_hbm.at[didx[b].at[isl]], drows[h],
                                  sem_gd[h]),
        )

    def out_cp(p, bp):
        return pltpu.make_async_copy(
            obuf[bp], out_hbm.at[:, pl.ds(base(p), B)], sem_o[bp])

    def compute_half(h, ob, col0):
        def grp(eg, c2):
            e0 = lax.iota(jnp.int32, D) + eg * D
            svecs = [plsc.load_gather(rows[h],
                                      [e0, jnp.full((D,), i, jnp.int32)])
                     for i in range(D)]
            dvecs = [plsc.load_gather(drows[h],
                                      [e0, jnp.full((D,), j, jnp.int32)])
                     for j in range(D)]
            for i in range(D):
                for j in range(D):
                    ob[i * D + j, pl.ds(col0 + eg * D, D)] = \
                        svecs[i] * dvecs[j]
            return c2
        lax.fori_loop(0, H // D, grp, 0, unroll=False)

    def start2(cps):
        cps[0].start()
        cps[1].start()

    def wait2(cps):
        cps[0].wait()
        cps[1].wait()

    cp0 = idx_cp(0, 0)
    start2(cp0)
    wait2(cp0)
    start2(gather(0, 0))
    start2(idx_cp(1, 1))

    def pair_step(p, bp, prefetch):
        start2(gather(bp, 1))
        if prefetch:
            start2(idx_cp(p + 1, 1 - bp))
        wait2(gather(bp, 0))
        pl.when(p >= 2)(lambda: out_cp(p - 2, bp).wait())
        compute_half(0, obuf[bp], 0)
        wait2(gather(bp, 1))
        compute_half(1, obuf[bp], H)
        out_cp(p, bp).start()
        if prefetch:
            wait2(idx_cp(p + 1, 1 - bp))
            start2(gather(1 - bp, 0))

    def quad(qq, c):
        pair_step(2 * qq, 0, True)
        pair_step(2 * qq + 1, 1, True)
        return c

    lax.fori_loop(0, (NP0 - 1) // 2, quad, 0, unroll=False)

    p_pen = NP0 - 1
    start2(gather(0, 1))
    pl.when(wid < NEXTRA)(lambda: start2(idx_cp(NP0, 1)))
    wait2(gather(0, 0))
    out_cp(p_pen - 2, 0).wait()
    compute_half(0, obuf[0], 0)
    wait2(gather(0, 1))
    compute_half(1, obuf[0], H)
    out_cp(p_pen, 0).start()

    @pl.when(wid < NEXTRA)
    def _extra():
        wait2(idx_cp(NP0, 1))
        start2(gather(1, 0))
        start2(gather(1, 1))
        wait2(gather(1, 0))
        out_cp(p_pen - 1, 1).wait()
        compute_half(0, obuf[1], 0)
        wait2(gather(1, 1))
        compute_half(1, obuf[1], H)
        out_cp(NP0, 1).start()
        out_cp(NP0, 1).wait()

    pl.when(wid >= NEXTRA)(lambda: out_cp(p_pen - 1, 1).wait())
    out_cp(p_pen, 0).wait()


def kernel(z, edge_label_index):
    e = edge_label_index.shape[1]
    src = edge_label_index[0]
    dst = edge_label_index[1]
    zt = jnp.pad(z, ((0, 0), (0, 128 - D)))
    mesh = plsc.VectorSubcoreMesh(core_axis_name="c", subcore_axis_name="s")
    ot = pl.kernel(
        _sc_body,
        out_type=jax.ShapeDtypeStruct((D * D, e), jnp.float32),
        mesh=mesh,
        scratch_types=[
            pltpu.VMEM((B,), jnp.int32),
            pltpu.VMEM((B,), jnp.int32),
            pltpu.VMEM((B,), jnp.int32),
            pltpu.VMEM((B,), jnp.int32),
            pltpu.VMEM((H, 128), jnp.float32),
            pltpu.VMEM((H, 128), jnp.float32),
            pltpu.VMEM((H, 128), jnp.float32),
            pltpu.VMEM((H, 128), jnp.float32),
            pltpu.VMEM((D * D, B), jnp.float32),
            pltpu.VMEM((D * D, B), jnp.float32),
            pltpu.SemaphoreType.DMA,
            pltpu.SemaphoreType.DMA,
            pltpu.SemaphoreType.DMA,
            pltpu.SemaphoreType.DMA,
            pltpu.SemaphoreType.DMA,
            pltpu.SemaphoreType.DMA,
            pltpu.SemaphoreType.DMA,
            pltpu.SemaphoreType.DMA,
            pltpu.SemaphoreType.DMA,
            pltpu.SemaphoreType.DMA,
        ],
        compiler_params=pltpu.CompilerParams(needs_layout_passes=False),
    )(zt, src, dst)
    return jnp.transpose(ot.reshape(1, D, D, e), (3, 0, 1, 2))


# speedup vs baseline: 7.5337x; 1.7183x over previous
import jax
import jax.numpy as jnp
from jax import lax
from jax.experimental import pallas as pl
from jax.experimental.pallas import tpu as pltpu
from jax.experimental.pallas import tpu_sc as plsc

D = 16
B = 128
H = 64
NC, NS = 2, 16
NW = NC * NS
NBLK = 800000 // B
NP0 = NBLK // NW
NEXTRA = NBLK - NP0 * NW


def _sc_body(z_hbm, eli_hbm, out_hbm,
             sidx0, sidx1, didx0, didx1, sr0, sr1, dr0, dr1, ob0, ob1,
             sem_is0, sem_is1, sem_id0, sem_id1,
             sem_gs0, sem_gs1, sem_gd0, sem_gd1,
             sem_o0, sem_o1):
    sidx = (sidx0, sidx1)
    didx = (didx0, didx1)
    rows = (sr0, sr1)
    drows = (dr0, dr1)
    obuf = (ob0, ob1)
    sem_is = (sem_is0, sem_is1)
    sem_id = (sem_id0, sem_id1)
    sem_gs = (sem_gs0, sem_gs1)
    sem_gd = (sem_gd0, sem_gd1)
    sem_o = (sem_o0, sem_o1)

    wid = lax.axis_index("s") * NC + lax.axis_index("c")

    def blk(p):
        return wid + p * NW

    def idx_cp(p, b):
        esl = pl.ds(blk(p) * B, B)
        return (
            pltpu.make_async_copy(eli_hbm.at[0, esl], sidx[b], sem_is[b]),
            pltpu.make_async_copy(eli_hbm.at[1, esl], didx[b], sem_id[b]),
        )

    def gather(b, h):
        isl = pl.ds(h * H, H)
        return (
            pltpu.make_async_copy(z_hbm.at[sidx[b].at[isl]], rows[h],
                                  sem_gs[h]),
            pltpu.make_async_copy(z_hbm.at[didx[b].at[isl]], drows[h],
                                  sem_gd[h]),
        )

    def out_cp(p, bp):
        return pltpu.make_async_copy(
            obuf[bp], out_hbm.at[:, blk(p)], sem_o[bp])

    def compute_half(h, ob, col0):
        def grp(eg, c2):
            e0 = lax.iota(jnp.int32, D) + eg * D
            svecs = [plsc.load_gather(rows[h],
                                      [e0, jnp.full((D,), i, jnp.int32)])
                     for i in range(D)]
            dvecs = [plsc.load_gather(drows[h],
                                      [e0, jnp.full((D,), j, jnp.int32)])
                     for j in range(D)]
            for i in range(D):
                for j in range(D):
                    k = i * D + j
                    ob[k // 8, k % 8, pl.ds(col0 + eg * D, D)] = \
                        svecs[i] * dvecs[j]
            return c2
        lax.fori_loop(0, H // D, grp, 0, unroll=False)

    def start2(cps):
        cps[0].start()
        cps[1].start()

    def wait2(cps):
        cps[0].wait()
        cps[1].wait()

    cp0 = idx_cp(0, 0)
    start2(cp0)
    wait2(cp0)
    start2(gather(0, 0))
    start2(idx_cp(1, 1))

    def pair_step(p, bp, prefetch):
        start2(gather(bp, 1))
        if prefetch:
            start2(idx_cp(p + 1, 1 - bp))
        wait2(gather(bp, 0))
        pl.when(p >= 2)(lambda: out_cp(p - 2, bp).wait())
        compute_half(0, obuf[bp], 0)
        wait2(gather(bp, 1))
        compute_half(1, obuf[bp], H)
        out_cp(p, bp).start()
        if prefetch:
            wait2(idx_cp(p + 1, 1 - bp))
            start2(gather(1 - bp, 0))

    def quad(qq, c):
        pair_step(2 * qq, 0, True)
        pair_step(2 * qq + 1, 1, True)
        return c

    lax.fori_loop(0, (NP0 - 1) // 2, quad, 0, unroll=False)

    p_pen = NP0 - 1
    start2(gather(0, 1))
    pl.when(wid < NEXTRA)(lambda: start2(idx_cp(NP0, 1)))
    wait2(gather(0, 0))
    out_cp(p_pen - 2, 0).wait()
    compute_half(0, obuf[0], 0)
    wait2(gather(0, 1))
    compute_half(1, obuf[0], H)
    out_cp(p_pen, 0).start()

    @pl.when(wid < NEXTRA)
    def _extra():
        wait2(idx_cp(NP0, 1))
        start2(gather(1, 0))
        start2(gather(1, 1))
        wait2(gather(1, 0))
        out_cp(p_pen - 1, 1).wait()
        compute_half(0, obuf[1], 0)
        wait2(gather(1, 1))
        compute_half(1, obuf[1], H)
        out_cp(NP0, 1).start()
        out_cp(NP0, 1).wait()

    pl.when(wid >= NEXTRA)(lambda: out_cp(p_pen - 1, 1).wait())
    out_cp(p_pen, 0).wait()


def kernel(z, edge_label_index):
    e = edge_label_index.shape[1]
    mesh = plsc.VectorSubcoreMesh(core_axis_name="c", subcore_axis_name="s")
    ot = pl.kernel(
        _sc_body,
        out_type=jax.ShapeDtypeStruct((32, NBLK, 8, B), jnp.float32),
        mesh=mesh,
        scratch_types=[
            pltpu.VMEM((B,), jnp.int32),
            pltpu.VMEM((B,), jnp.int32),
            pltpu.VMEM((B,), jnp.int32),
            pltpu.VMEM((B,), jnp.int32),
            pltpu.VMEM((H, D), jnp.float32),
            pltpu.VMEM((H, D), jnp.float32),
            pltpu.VMEM((H, D), jnp.float32),
            pltpu.VMEM((H, D), jnp.float32),
            pltpu.VMEM((32, 8, B), jnp.float32),
            pltpu.VMEM((32, 8, B), jnp.float32),
            pltpu.SemaphoreType.DMA,
            pltpu.SemaphoreType.DMA,
            pltpu.SemaphoreType.DMA,
            pltpu.SemaphoreType.DMA,
            pltpu.SemaphoreType.DMA,
            pltpu.SemaphoreType.DMA,
            pltpu.SemaphoreType.DMA,
            pltpu.SemaphoreType.DMA,
            pltpu.SemaphoreType.DMA,
            pltpu.SemaphoreType.DMA,
        ],
        compiler_params=pltpu.CompilerParams(
            use_tc_tiling_on_sc=False, needs_layout_passes=False),
    )(z, edge_label_index)
    kt = jnp.transpose(ot, (0, 2, 1, 3)).reshape(1, D, D, e)
    return jnp.transpose(kt, (3, 0, 1, 2))
